# baseline (speedup 1.0000x reference)
import jax
import jax.numpy as jnp
from jax.experimental import pallas as pl
from jax.experimental.pallas import tpu as pltpu

_N_E = 1024
_E_DIM = 64
_BETA = 0.25
_BPS = 8


def _vq_body(z_ref, emb_ref, en2_ref, zq_ref, idx_ref, loss_ref):
    emb = emb_ref[...]
    en2c = en2_ref[...]

    part = jnp.float32(0.0)
    for b in range(_BPS):
        z_blk = z_ref[b]

        scores = jax.lax.dot_general(
            emb, z_blk,
            dimension_numbers=(((1,), (0,)), ((), ())),
            preferred_element_type=jnp.float32)

        zn2 = jnp.sum(z_blk * z_blk, axis=0)
        d = (zn2[None, :] + en2c) - 2.0 * scores
        idx = jnp.argmin(d, axis=0).astype(jnp.int32)

        row = jax.lax.broadcasted_iota(jnp.int32, d.shape, 0)
        onehot = (row == idx[None, :]).astype(jnp.float32)
        zq = jax.lax.dot_general(
            emb, onehot,
            dimension_numbers=(((0,), (0,)), ((), ())),
            preferred_element_type=jnp.float32)

        zq_ref[b] = zq
        idx_ref[b, 0] = idx

        diff = zq - z_blk
        part = part + jnp.sum(diff * diff)

    @pl.when(pl.program_id(0) == 0)
    def _init():
        loss_ref[0, 0] = 0.0

    loss_ref[0, 0] += part


@jax.jit
def kernel(z, embedding):
    B, C, H, W = z.shape
    HW = H * W
    z3 = z.reshape(B, C, HW)
    en2 = jnp.sum(embedding ** 2, axis=1)[:, None]

    zq, idx, loss_sum = pl.pallas_call(
        _vq_body,
        grid=(B // _BPS,),
        in_specs=[
            pl.BlockSpec((_BPS, C, HW), lambda b: (b, 0, 0)),
            pl.BlockSpec((_N_E, _E_DIM), lambda b: (0, 0)),
            pl.BlockSpec((_N_E, 1), lambda b: (0, 0)),
        ],
        out_specs=[
            pl.BlockSpec((_BPS, C, HW), lambda b: (b, 0, 0)),
            pl.BlockSpec((_BPS, 1, HW), lambda b: (b, 0, 0)),
            pl.BlockSpec(memory_space=pltpu.SMEM),
        ],
        out_shape=[
            jax.ShapeDtypeStruct((B, C, HW), jnp.float32),
            jax.ShapeDtypeStruct((B, 1, HW), jnp.int32),
            jax.ShapeDtypeStruct((1, 1), jnp.float32),
        ],
    )(z3, embedding, en2)

    loss = _BETA * loss_sum[0, 0] / (B * C * H * W)
    return (zq.reshape(B, C, H, W), loss, idx.reshape(B, H, W))


# speedup vs baseline: 1.0092x; 1.0092x over previous
import jax
import jax.numpy as jnp
from jax.experimental import pallas as pl
from jax.experimental.pallas import tpu as pltpu

_N_E = 1024
_E_DIM = 64
_BETA = 0.25
_BPS = 8


def _vq_body(z_ref, emb_ref, zq_ref, idx_ref, loss_ref):
    emb = emb_ref[...]
    en2 = jnp.sum(emb * emb, axis=1)

    part = jnp.float32(0.0)
    for b in range(_BPS):
        z_blk = z_ref[b]

        scores = jax.lax.dot_general(
            emb, z_blk,
            dimension_numbers=(((1,), (0,)), ((), ())),
            preferred_element_type=jnp.float32)

        zn2 = jnp.sum(z_blk * z_blk, axis=0)
        d = (zn2[None, :] + en2[:, None]) - 2.0 * scores
        idx = jnp.argmin(d, axis=0).astype(jnp.int32)

        row = jax.lax.broadcasted_iota(jnp.int32, d.shape, 0)
        onehot = (row == idx[None, :]).astype(jnp.float32)
        zq = jax.lax.dot_general(
            emb, onehot,
            dimension_numbers=(((0,), (0,)), ((), ())),
            preferred_element_type=jnp.float32)

        zq_ref[b] = zq
        idx_ref[b, 0] = idx

        diff = zq - z_blk
        part = part + jnp.sum(diff * diff)

    @pl.when(pl.program_id(0) == 0)
    def _init():
        loss_ref[0, 0] = 0.0

    loss_ref[0, 0] += part


@jax.jit
def kernel(z, embedding):
    B, C, H, W = z.shape
    HW = H * W
    z3 = z.reshape(B, C, HW)

    zq, idx, loss_sum = pl.pallas_call(
        _vq_body,
        grid=(B // _BPS,),
        in_specs=[
            pl.BlockSpec((_BPS, C, HW), lambda b: (b, 0, 0)),
            pl.BlockSpec((_N_E, _E_DIM), lambda b: (0, 0)),
        ],
        out_specs=[
            pl.BlockSpec((_BPS, C, HW), lambda b: (b, 0, 0)),
            pl.BlockSpec((_BPS, 1, HW), lambda b: (b, 0, 0)),
            pl.BlockSpec(memory_space=pltpu.SMEM),
        ],
        out_shape=[
            jax.ShapeDtypeStruct((B, C, HW), jnp.float32),
            jax.ShapeDtypeStruct((B, 1, HW), jnp.int32),
            jax.ShapeDtypeStruct((1, 1), jnp.float32),
        ],
    )(z3, embedding)

    loss = _BETA * loss_sum[0, 0] / (B * C * H * W)
    return (zq.reshape(B, C, H, W), loss, idx.reshape(B, H, W))
